# Initial kernel scaffold; baseline (speedup 1.0000x reference)
import jax
import jax.numpy as jnp
from jax.experimental import pallas as pl


def kernel(edge_index, neighbor, qry_embeds, proto_embeds, msg_W1, msg_b1, msg_W2, msg_b2, trans_W1, trans_b1, trans_W2, trans_b2, trans_W3):
    raise NotImplementedError("write your pallas kernel here")


# speedup vs baseline: 3.6322x; 3.6322x over previous
import functools

import jax
import jax.numpy as jnp
from jax import lax
from jax.experimental import pallas as pl
from jax.experimental.pallas import tpu as pltpu
from jax.experimental.pallas import tpu_sc as plsc

NC = 2
NS = 16
NW = NC * NS
C = 80
RAW = 128
CNT_W = 16


def _sc_gather(x, xn, row, col):
    E = row.shape[0]
    per_w = E // NW
    n_chunks = per_w // C
    mesh = plsc.VectorSubcoreMesh(
        core_axis_name="c", subcore_axis_name="s", num_cores=NC,
        num_subcores=NS)
    fdt = jax.ShapeDtypeStruct((E, RAW), jnp.float32)

    @functools.partial(
        pl.kernel, mesh=mesh,
        out_type=(fdt, fdt, fdt),
        scratch_types=[
            pltpu.VMEM((C,), jnp.int32),
            pltpu.VMEM((C,), jnp.int32),
            pltpu.VMEM((C, RAW), jnp.float32),
            pltpu.VMEM((C, RAW), jnp.float32),
            pltpu.VMEM((C, RAW), jnp.float32),
            pltpu.SemaphoreType.DMA,
            pltpu.SemaphoreType.DMA,
            pltpu.SemaphoreType.DMA,
        ],
    )
    def k(x_hbm, xn_hbm, row_hbm, col_hbm, xr_out, xc_out, xnc_out,
          idx_r, idx_c, xr_v, xc_v, xnc_v, sem0, sem1, sem2):
        wid = lax.axis_index("s") * NC + lax.axis_index("c")
        w_base = wid * per_w

        def body(kk, _):
            base = w_base + kk * C
            pltpu.sync_copy(row_hbm.at[pl.ds(base, C)], idx_r)
            pltpu.sync_copy(col_hbm.at[pl.ds(base, C)], idx_c)
            d0 = pltpu.async_copy(x_hbm.at[idx_r], xr_v, sem0)
            d1 = pltpu.async_copy(x_hbm.at[idx_c], xc_v, sem1)
            d2 = pltpu.async_copy(xn_hbm.at[idx_c], xnc_v, sem2)
            d0.wait()
            d1.wait()
            d2.wait()
            pltpu.sync_copy(xr_v, xr_out.at[pl.ds(base, C)])
            pltpu.sync_copy(xc_v, xc_out.at[pl.ds(base, C)])
            pltpu.sync_copy(xnc_v, xnc_out.at[pl.ds(base, C)])
            return _

        lax.fori_loop(0, n_chunks, body, None)

    return k(x, xn, row, col)


def _tc_mlp(xr, xc, xnc, w1a, w1b, b1, w2, b2, tw1, tb1, tw2, tb2, tw3):
    E = xr.shape[0]
    B = 2000
    grid = (E // B,)

    def body(xr_ref, xc_ref, xnc_ref, w1a_ref, w1b_ref, b1_ref, w2_ref,
             b2_ref, tw1_ref, tb1_ref, tw2_ref, tb2_ref, tw3_ref, out_ref):
        diff = xr_ref[...] - xc_ref[...]
        sqd = jnp.sum(diff * diff, axis=1, keepdims=True)
        h = jnp.dot(xnc_ref[...], w1a_ref[...],
                    preferred_element_type=jnp.float32)
        h = h + sqd * w1b_ref[...] + b1_ref[...]
        h = h * jax.nn.sigmoid(h)
        h = jnp.dot(h, w2_ref[...], preferred_element_type=jnp.float32)
        h = h + b2_ref[...]
        h = h * jax.nn.sigmoid(h)
        h = jnp.dot(h, tw1_ref[...], preferred_element_type=jnp.float32)
        h = h + tb1_ref[...]
        h = h * jax.nn.sigmoid(h)
        h = jnp.dot(h, tw2_ref[...], preferred_element_type=jnp.float32)
        h = h + tb2_ref[...]
        h = h * jax.nn.sigmoid(h)
        t = jnp.sum(h * tw3_ref[...], axis=1, keepdims=True)
        out_ref[...] = diff * t

    blk_e = pl.BlockSpec((B, RAW), lambda i: (i, 0))
    full = lambda shape: pl.BlockSpec(shape, lambda i: tuple(0 for _ in shape))
    return pl.pallas_call(
        body,
        grid=grid,
        in_specs=[
            blk_e, blk_e, blk_e,
            full((RAW, 64)), full((1, 64)), full((1, 64)),
            full((64, 64)), full((1, 64)),
            full((64, 64)), full((1, 64)),
            full((64, 64)), full((1, 64)),
            full((1, 64)),
        ],
        out_specs=blk_e,
        out_shape=jax.ShapeDtypeStruct((E, RAW), jnp.float32),
    )(xr, xc, xnc, w1a, w1b, b1, w2, b2, tw1, tb1, tw2, tb2, tw3)


def _sc_scatter(trans, row, zrow, ones_c, n_proto, NQ):
    E, _ = trans.shape
    per_w = E // NW
    n_chunks = per_w // C
    rows_per_tile = NQ // NS
    n_init = rows_per_tile // C
    trash = NQ - 1
    mesh = plsc.VectorSubcoreMesh(
        core_axis_name="c", subcore_axis_name="s", num_cores=NC,
        num_subcores=NS)
    fdt = jax.ShapeDtypeStruct((NC, NQ, RAW), jnp.float32)

    @functools.partial(
        pl.kernel, mesh=mesh,
        out_type=(fdt, fdt),
        scratch_types=[
            pltpu.VMEM_SHARED((NQ, RAW), jnp.float32),
            pltpu.VMEM_SHARED((NQ, RAW), jnp.float32),
            pltpu.VMEM((C,), jnp.int32),
            pltpu.VMEM((C, RAW), jnp.float32),
            pltpu.VMEM((C, RAW), jnp.float32),
        ],
    )
    def k(trans_hbm, row_hbm, zrow_hbm, ones_hbm, s_out, cnt_out,
          acc_sh, cnt_sh, idx_v, tr_v, ones_v):
        cid = lax.axis_index("c")
        sid = lax.axis_index("s")
        wid = sid * NC + cid
        w_base = wid * per_w
        tile_rows = sid * rows_per_tile

        pltpu.sync_copy(zrow_hbm, tr_v)

        def zbody(j, _):
            pltpu.sync_copy(tr_v, acc_sh.at[pl.ds(tile_rows + j * C, C)])
            pltpu.sync_copy(tr_v, cnt_sh.at[pl.ds(tile_rows + j * C, C)])
            return _

        lax.fori_loop(0, n_init, zbody, None)
        pltpu.sync_copy(ones_hbm, ones_v)
        plsc.subcore_barrier()

        def body(kk, _):
            base = w_base + kk * C
            pltpu.sync_copy(row_hbm.at[pl.ds(base, C)], idx_v)
            pltpu.sync_copy(trans_hbm.at[pl.ds(base, C)], tr_v)
            for j in range(C // 16):
                v = idx_v[pl.ds(j * 16, 16)]
                idx_v[pl.ds(j * 16, 16)] = jnp.where(
                    v >= n_proto, v - n_proto, trash)
            pltpu.sync_copy(tr_v, acc_sh.at[idx_v], add=True)
            pltpu.sync_copy(ones_v, cnt_sh.at[idx_v], add=True)
            return _

        lax.fori_loop(0, n_chunks, body, None)
        plsc.subcore_barrier()

        def obody(j, _):
            r = tile_rows + j * C
            pltpu.sync_copy(acc_sh.at[pl.ds(r, C)], tr_v)
            pltpu.sync_copy(tr_v, s_out.at[cid, pl.ds(r, C)])
            pltpu.sync_copy(cnt_sh.at[pl.ds(r, C)], ones_v)
            pltpu.sync_copy(ones_v, cnt_out.at[cid, pl.ds(r, C)])
            return _

        lax.fori_loop(0, n_init, obody, None)

    return k(trans, row, zrow, ones_c)


def _tc_finish(qry, s_parts, cnt_parts):
    nq = qry.shape[0]

    def body(qry_ref, s_ref, cnt_ref, out_ref):
        s = s_ref[0] + s_ref[1]
        cnt = cnt_ref[0, :, 0:1] + cnt_ref[1, :, 0:1]
        out_ref[...] = qry_ref[...] + s / jnp.maximum(cnt, 1.0)

    return pl.pallas_call(
        body,
        grid=(1,),
        in_specs=[
            pl.BlockSpec((nq, RAW), lambda i: (0, 0)),
            pl.BlockSpec((NC, nq, RAW), lambda i: (0, 0, 0)),
            pl.BlockSpec((NC, nq, RAW), lambda i: (0, 0, 0)),
        ],
        out_specs=pl.BlockSpec((nq, RAW), lambda i: (0, 0)),
        out_shape=jax.ShapeDtypeStruct((nq, RAW), jnp.float32),
    )(qry, s_parts, cnt_parts)


def kernel(edge_index, neighbor, qry_embeds, proto_embeds,
           msg_W1, msg_b1, msg_W2, msg_b2,
           trans_W1, trans_b1, trans_W2, trans_b2, trans_W3):
    n_proto = proto_embeds.shape[0]
    x = jnp.concatenate([proto_embeds, qry_embeds], axis=0)
    xn = jnp.concatenate([proto_embeds, neighbor], axis=0)
    row = edge_index[0]
    col = edge_index[1]
    N = x.shape[0]

    xr, xc, xnc = _sc_gather(x, xn, row, col)

    w1a = msg_W1[:RAW]
    w1b = msg_W1[RAW:RAW + 1]
    trans = _tc_mlp(xr, xc, xnc, w1a, w1b, msg_b1.reshape(1, -1),
                    msg_W2, msg_b2.reshape(1, -1),
                    trans_W1, trans_b1.reshape(1, -1),
                    trans_W2, trans_b2.reshape(1, -1),
                    trans_W3.reshape(1, -1))

    nq = qry_embeds.shape[0]
    nq_pad = ((nq + 1 + NS * C - 1) // (NS * C)) * (NS * C)
    zrow = jnp.zeros((C, RAW), jnp.float32)
    ones_c = jnp.ones((C, RAW), jnp.float32)
    s_parts, cnt_parts = _sc_scatter(trans, row, zrow, ones_c, n_proto, nq_pad)

    qry_new = _tc_finish(qry_embeds, s_parts, cnt_parts)
    return (neighbor, qry_new)


# speedup vs baseline: 3.9869x; 1.0977x over previous
import functools

import jax
import jax.numpy as jnp
from jax import lax
from jax.experimental import pallas as pl
from jax.experimental.pallas import tpu as pltpu
from jax.experimental.pallas import tpu_sc as plsc

NC = 2
NS = 16
NW = NC * NS
C = 40
RAW = 128


def _sc_gather(x, xn, row, col):
    E = row.shape[0]
    per_w = E // NW
    n_pairs = per_w // (2 * C)
    mesh = plsc.VectorSubcoreMesh(
        core_axis_name="c", subcore_axis_name="s", num_cores=NC,
        num_subcores=NS)
    fdt = jax.ShapeDtypeStruct((E, RAW), jnp.float32)

    @functools.partial(
        pl.kernel, mesh=mesh,
        out_type=(fdt, fdt, fdt),
        scratch_types=[
            pltpu.VMEM((C,), jnp.int32),
            pltpu.VMEM((C,), jnp.int32),
            pltpu.VMEM((C,), jnp.int32),
            pltpu.VMEM((C,), jnp.int32),
            pltpu.VMEM((C, RAW), jnp.float32),
            pltpu.VMEM((C, RAW), jnp.float32),
            pltpu.VMEM((C, RAW), jnp.float32),
            pltpu.VMEM((C, RAW), jnp.float32),
            pltpu.VMEM((C, RAW), jnp.float32),
            pltpu.VMEM((C, RAW), jnp.float32),
            pltpu.SemaphoreType.DMA,
            pltpu.SemaphoreType.DMA,
        ],
    )
    def k(x_hbm, xn_hbm, row_hbm, col_hbm, xr_out, xc_out, xnc_out,
          ir_a, ic_a, ir_b, ic_b, xr_a, xc_a, xnc_a, xr_b, xc_b, xnc_b,
          sem_a, sem_b):
        wid = lax.axis_index("s") * NC + lax.axis_index("c")
        w_base = wid * per_w

        def drain_b():
            pltpu.make_async_copy(x_hbm.at[pl.ds(0, C)], xr_b, sem_b).wait()
            pltpu.make_async_copy(x_hbm.at[pl.ds(0, C)], xc_b, sem_b).wait()
            pltpu.make_async_copy(x_hbm.at[pl.ds(0, C)], xnc_b, sem_b).wait()

        def body(i, _):
            a = w_base + (2 * i) * C
            b = a + C
            pltpu.sync_copy(row_hbm.at[pl.ds(a, C)], ir_a)
            pltpu.sync_copy(col_hbm.at[pl.ds(a, C)], ic_a)
            da0 = pltpu.async_copy(x_hbm.at[ir_a], xr_a, sem_a)
            da1 = pltpu.async_copy(x_hbm.at[ic_a], xc_a, sem_a)
            da2 = pltpu.async_copy(xn_hbm.at[ic_a], xnc_a, sem_a)

            @pl.when(i > 0)
            def _prev():
                bp = a - C
                drain_b()
                pltpu.sync_copy(xr_b, xr_out.at[pl.ds(bp, C)])
                pltpu.sync_copy(xc_b, xc_out.at[pl.ds(bp, C)])
                pltpu.sync_copy(xnc_b, xnc_out.at[pl.ds(bp, C)])

            pltpu.sync_copy(row_hbm.at[pl.ds(b, C)], ir_b)
            pltpu.sync_copy(col_hbm.at[pl.ds(b, C)], ic_b)
            pltpu.async_copy(x_hbm.at[ir_b], xr_b, sem_b)
            pltpu.async_copy(x_hbm.at[ic_b], xc_b, sem_b)
            pltpu.async_copy(xn_hbm.at[ic_b], xnc_b, sem_b)
            da0.wait()
            da1.wait()
            da2.wait()
            pltpu.sync_copy(xr_a, xr_out.at[pl.ds(a, C)])
            pltpu.sync_copy(xc_a, xc_out.at[pl.ds(a, C)])
            pltpu.sync_copy(xnc_a, xnc_out.at[pl.ds(a, C)])
            return _

        lax.fori_loop(0, n_pairs, body, None)
        bl = w_base + per_w - C
        drain_b()
        pltpu.sync_copy(xr_b, xr_out.at[pl.ds(bl, C)])
        pltpu.sync_copy(xc_b, xc_out.at[pl.ds(bl, C)])
        pltpu.sync_copy(xnc_b, xnc_out.at[pl.ds(bl, C)])

    return k(x, xn, row, col)


def _tc_mlp(xr, xc, xnc, row3, n_proto, trash,
            w1a, w1b, b1, w2, b2, tw1, tb1, tw2, tb2, tw3):
    E = xr.shape[0]
    B = 2000
    grid = (E // B,)

    def body(xr_ref, xc_ref, xnc_ref, row_ref, w1a_ref, w1b_ref, b1_ref,
             w2_ref, b2_ref, tw1_ref, tb1_ref, tw2_ref, tb2_ref, tw3_ref,
             out_ref, rowq_ref):
        r = row_ref[0, 0, :]
        rowq_ref[0, 0, :] = jnp.where(r >= n_proto, r - n_proto, trash)
        diff = xr_ref[...] - xc_ref[...]
        sqd = jnp.sum(diff * diff, axis=1, keepdims=True)
        h = jnp.dot(xnc_ref[...], w1a_ref[...],
                    preferred_element_type=jnp.float32)
        h = h + sqd * w1b_ref[...] + b1_ref[...]
        h = h * jax.nn.sigmoid(h)
        h = jnp.dot(h, w2_ref[...], preferred_element_type=jnp.float32)
        h = h + b2_ref[...]
        h = h * jax.nn.sigmoid(h)
        h = jnp.dot(h, tw1_ref[...], preferred_element_type=jnp.float32)
        h = h + tb1_ref[...]
        h = h * jax.nn.sigmoid(h)
        h = jnp.dot(h, tw2_ref[...], preferred_element_type=jnp.float32)
        h = h + tb2_ref[...]
        h = h * jax.nn.sigmoid(h)
        t = jnp.sum(h * tw3_ref[...], axis=1, keepdims=True)
        out_ref[...] = diff * t

    blk_e = pl.BlockSpec((B, RAW), lambda i: (i, 0))
    blk_r = pl.BlockSpec((1, 1, B), lambda i: (i, 0, 0))
    full = lambda shape: pl.BlockSpec(shape, lambda i: tuple(0 for _ in shape))
    return pl.pallas_call(
        body,
        grid=grid,
        in_specs=[
            blk_e, blk_e, blk_e, blk_r,
            full((RAW, 64)), full((1, 64)), full((1, 64)),
            full((64, 64)), full((1, 64)),
            full((64, 64)), full((1, 64)),
            full((64, 64)), full((1, 64)),
            full((1, 64)),
        ],
        out_specs=(blk_e, blk_r),
        out_shape=(jax.ShapeDtypeStruct((E, RAW), jnp.float32),
                   jax.ShapeDtypeStruct((E // B, 1, B), jnp.int32)),
    )(xr, xc, xnc, row3, w1a, w1b, b1, w2, b2, tw1, tb1, tw2, tb2, tw3)


def _sc_scatter(trans, rowq, zrow, ones_c, NQ):
    E, _ = trans.shape
    per_w = E // NW
    n_chunks = per_w // C
    rows_per_tile = NQ // NS
    n_init = rows_per_tile // C
    mesh = plsc.VectorSubcoreMesh(
        core_axis_name="c", subcore_axis_name="s", num_cores=NC,
        num_subcores=NS)
    fdt = jax.ShapeDtypeStruct((NC, NQ, RAW), jnp.float32)

    n_pairs = n_chunks // 2

    @functools.partial(
        pl.kernel, mesh=mesh,
        out_type=(fdt, fdt),
        scratch_types=[
            pltpu.VMEM_SHARED((NQ, RAW), jnp.float32),
            pltpu.VMEM_SHARED((NQ, RAW), jnp.float32),
            pltpu.VMEM((C,), jnp.int32),
            pltpu.VMEM((C,), jnp.int32),
            pltpu.VMEM((C, RAW), jnp.float32),
            pltpu.VMEM((C, RAW), jnp.float32),
            pltpu.VMEM((C, RAW), jnp.float32),
            pltpu.SemaphoreType.DMA,
            pltpu.SemaphoreType.DMA,
        ],
    )
    def k(trans_hbm, rowq_hbm, zrow_hbm, ones_hbm, s_out, cnt_out,
          acc_sh, cnt_sh, idx_a, idx_b, tr_a, tr_b, ones_v, sem_a, sem_b):
        cid = lax.axis_index("c")
        sid = lax.axis_index("s")
        wid = sid * NC + cid
        w_base = wid * per_w
        tile_rows = sid * rows_per_tile

        pltpu.sync_copy(zrow_hbm, tr_a)

        def zbody(j, _):
            pltpu.sync_copy(tr_a, acc_sh.at[pl.ds(tile_rows + j * C, C)])
            pltpu.sync_copy(tr_a, cnt_sh.at[pl.ds(tile_rows + j * C, C)])
            return _

        lax.fori_loop(0, n_init, zbody, None)
        pltpu.sync_copy(ones_hbm, ones_v)
        plsc.subcore_barrier()

        def scat_b():
            pltpu.make_async_copy(
                trans_hbm.at[pl.ds(0, C)], tr_b, sem_b).wait()
            pltpu.sync_copy(tr_b, acc_sh.at[idx_b], add=True)
            pltpu.sync_copy(ones_v, cnt_sh.at[idx_b], add=True)

        def body(i, _):
            a = w_base + (2 * i) * C
            b = a + C
            pltpu.sync_copy(rowq_hbm.at[pl.ds(a, C)], idx_a)
            da = pltpu.async_copy(trans_hbm.at[pl.ds(a, C)], tr_a, sem_a)

            @pl.when(i > 0)
            def _prev():
                scat_b()

            pltpu.sync_copy(rowq_hbm.at[pl.ds(b, C)], idx_b)
            pltpu.async_copy(trans_hbm.at[pl.ds(b, C)], tr_b, sem_b)
            da.wait()
            pltpu.sync_copy(tr_a, acc_sh.at[idx_a], add=True)
            pltpu.sync_copy(ones_v, cnt_sh.at[idx_a], add=True)
            return _

        lax.fori_loop(0, n_pairs, body, None)
        scat_b()
        plsc.subcore_barrier()

        def obody(j, _):
            r = tile_rows + j * C
            pltpu.sync_copy(acc_sh.at[pl.ds(r, C)], tr_a)
            pltpu.sync_copy(tr_a, s_out.at[cid, pl.ds(r, C)])
            pltpu.sync_copy(cnt_sh.at[pl.ds(r, C)], ones_v)
            pltpu.sync_copy(ones_v, cnt_out.at[cid, pl.ds(r, C)])
            return _

        lax.fori_loop(0, n_init, obody, None)

    return k(trans, rowq, zrow, ones_c)


def _tc_finish(qry, s_parts, cnt_parts):
    nq = qry.shape[0]

    def body(qry_ref, s_ref, cnt_ref, out_ref):
        s = s_ref[0] + s_ref[1]
        cnt = cnt_ref[0, :, 0:1] + cnt_ref[1, :, 0:1]
        out_ref[...] = qry_ref[...] + s / jnp.maximum(cnt, 1.0)

    return pl.pallas_call(
        body,
        grid=(1,),
        in_specs=[
            pl.BlockSpec((nq, RAW), lambda i: (0, 0)),
            pl.BlockSpec((NC, nq, RAW), lambda i: (0, 0, 0)),
            pl.BlockSpec((NC, nq, RAW), lambda i: (0, 0, 0)),
        ],
        out_specs=pl.BlockSpec((nq, RAW), lambda i: (0, 0)),
        out_shape=jax.ShapeDtypeStruct((nq, RAW), jnp.float32),
    )(qry, s_parts, cnt_parts)


def kernel(edge_index, neighbor, qry_embeds, proto_embeds,
           msg_W1, msg_b1, msg_W2, msg_b2,
           trans_W1, trans_b1, trans_W2, trans_b2, trans_W3):
    n_proto = proto_embeds.shape[0]
    x = jnp.concatenate([proto_embeds, qry_embeds], axis=0)
    xn = jnp.concatenate([proto_embeds, neighbor], axis=0)
    E = edge_index.shape[1]

    xr, xc, xnc = _sc_gather(x, xn, edge_index[0], edge_index[1])

    nq = qry_embeds.shape[0]
    nq_pad = ((nq + 1 + NS * C - 1) // (NS * C)) * (NS * C)
    B = 2000
    row3 = edge_index[0].reshape(E // B, 1, B)
    w1a = msg_W1[:RAW]
    w1b = msg_W1[RAW:RAW + 1]
    trans, rowq3 = _tc_mlp(xr, xc, xnc, row3, n_proto, nq_pad - 1,
                           w1a, w1b, msg_b1.reshape(1, -1),
                           msg_W2, msg_b2.reshape(1, -1),
                           trans_W1, trans_b1.reshape(1, -1),
                           trans_W2, trans_b2.reshape(1, -1),
                           trans_W3.reshape(1, -1))

    zrow = jnp.zeros((C, RAW), jnp.float32)
    ones_c = jnp.ones((C, RAW), jnp.float32)
    s_parts, cnt_parts = _sc_scatter(trans, rowq3.reshape(E), zrow, ones_c,
                                     nq_pad)

    qry_new = _tc_finish(qry_embeds, s_parts, cnt_parts)
    return (neighbor, qry_new)
